# baseline (speedup 1.0000x reference)
import jax
import jax.numpy as jnp
from jax.experimental import pallas as pl
from jax.experimental.pallas import tpu as pltpu

H = 64
W = 64
C = 256
N = 4
PH = 7
PW = 7
SCALE = 64.0
KB = 16


def _axis_coords(lo, nbins, binsz, limit):
    shp = (nbins, KB, 2)
    pv = jax.lax.broadcasted_iota(jnp.int32, shp, 0).astype(jnp.float32)
    iv = jax.lax.broadcasted_iota(jnp.int32, shp, 2).astype(jnp.float32)
    lo_r = lo.reshape(1, KB, 1)
    bin_r = binsz.reshape(1, KB, 1)
    coord = lo_r + pv * bin_r + (iv + 0.5) * bin_r / 2.0
    valid = (coord >= -1.0) & (coord <= float(limit))
    cc = jnp.minimum(jnp.maximum(coord, 0.0), float(limit - 1))
    return cc, jnp.where(valid, 0.5, 0.0)


def _body(feat_ref, rois_ref, out_ref, t2_ref):
    r = rois_ref[...]
    batch = r[:, 0:1].astype(jnp.int32)
    x1 = r[:, 1:2] * SCALE
    y1 = r[:, 2:3] * SCALE
    x2 = r[:, 3:4] * SCALE
    y2 = r[:, 4:5] * SCALE
    roi_w = jnp.maximum(x2 - x1, 1.0)
    roi_h = jnp.maximum(y2 - y1, 1.0)
    bin_w = roi_w / float(PW)
    bin_h = roi_h / float(PH)

    yc, wvy = _axis_coords(y1, PH, bin_h, H)
    jy = jax.lax.broadcasted_iota(jnp.int32, (PH, KB, 2, N * H), 3)
    jyf = (jy - batch.reshape(1, KB, 1, 1) * H).astype(jnp.float32)
    wy4 = wvy[..., None] * jnp.maximum(1.0 - jnp.abs(yc[..., None] - jyf), 0.0)
    wy = wy4.sum(axis=2).reshape(PH * KB, N * H).astype(jnp.bfloat16)

    xc, wvx = _axis_coords(x1, PW, bin_w, W)
    q = jax.lax.broadcasted_iota(jnp.int32, (PW, KB, 2, W * KB), 3)
    xq = (q // KB).astype(jnp.float32)
    kq = q % KB
    kk = jax.lax.broadcasted_iota(jnp.int32, (PW, KB, 2, W * KB), 1)
    bd4 = wvx[..., None] * jnp.maximum(1.0 - jnp.abs(xc[..., None] - xq), 0.0)
    bd4 = jnp.where(kq == kk, bd4, 0.0)
    bd = bd4.sum(axis=2).reshape(PW * KB, W * KB).astype(jnp.bfloat16)

    for x0 in range(W):
        chunk = jnp.dot(wy, feat_ref[:, pl.ds(x0 * C, C)],
                        preferred_element_type=jnp.float32)
        t2_ref[x0] = chunk.astype(jnp.bfloat16)

    for ph in range(PH):
        rhs = t2_ref[:, pl.ds(ph * KB, KB), :].reshape(W * KB, C)
        out_ph = jnp.dot(bd, rhs, preferred_element_type=jnp.float32)
        out_ref[ph] = out_ph.astype(jnp.bfloat16).reshape(PW, KB, C)


def kernel(input_tensor, rois):
    k = rois.shape[0]
    kp = ((k + KB - 1) // KB) * KB
    if kp != k:
        rois = jnp.concatenate(
            [rois, jnp.zeros((kp - k, 5), rois.dtype)], axis=0)
    featT = jnp.transpose(input_tensor, (0, 2, 3, 1)).reshape(
        N * H, W * C).astype(jnp.bfloat16)
    grid = (kp // KB,)
    out = pl.pallas_call(
        _body,
        grid=grid,
        in_specs=[
            pl.BlockSpec((N * H, W * C), lambda i: (0, 0)),
            pl.BlockSpec((KB, 5), lambda i: (i, 0)),
        ],
        out_specs=pl.BlockSpec((PH, PW, KB, C), lambda i: (0, 0, i, 0)),
        out_shape=jax.ShapeDtypeStruct((PH, PW, kp, C), jnp.bfloat16),
        scratch_shapes=[pltpu.VMEM((W, PH * KB, C), jnp.bfloat16)],
        compiler_params=pltpu.CompilerParams(
            dimension_semantics=("arbitrary",),
        ),
    )(featT, rois)
    return jnp.transpose(out[:, :, :k], (2, 3, 0, 1)).astype(jnp.float32)


# speedup vs baseline: 1.0218x; 1.0218x over previous
import jax
import jax.numpy as jnp
from jax.experimental import pallas as pl
from jax.experimental.pallas import tpu as pltpu

H = 64
W = 64
C = 256
N = 4
PH = 7
PW = 7
SCALE = 64.0
KB = 16


def _axis_coords(lo, nbins, binsz, limit):
    shp = (nbins, KB, 2)
    pv = jax.lax.broadcasted_iota(jnp.int32, shp, 0).astype(jnp.float32)
    iv = jax.lax.broadcasted_iota(jnp.int32, shp, 2).astype(jnp.float32)
    lo_r = lo.reshape(1, KB, 1)
    bin_r = binsz.reshape(1, KB, 1)
    coord = lo_r + pv * bin_r + (iv + 0.5) * bin_r / 2.0
    valid = (coord >= -1.0) & (coord <= float(limit))
    cc = jnp.minimum(jnp.maximum(coord, 0.0), float(limit - 1))
    return cc, jnp.where(valid, 0.5, 0.0)


def _body(feat_ref, rois_ref, cy_ref, cxk_ref, out_ref, t2_ref):
    r = rois_ref[...]
    batch = r[:, 0:1].astype(jnp.int32)
    x1 = r[:, 1:2] * SCALE
    y1 = r[:, 2:3] * SCALE
    x2 = r[:, 3:4] * SCALE
    y2 = r[:, 4:5] * SCALE
    roi_w = jnp.maximum(x2 - x1, 1.0)
    roi_h = jnp.maximum(y2 - y1, 1.0)
    bin_w = roi_w / float(PW)
    bin_h = roi_h / float(PH)

    yc, wvy = _axis_coords(y1, PH, bin_h, H)
    ycb = yc + batch.reshape(1, KB, 1).astype(jnp.float32) * float(H)
    cy = cy_ref[...].reshape(1, 1, 1, N * H)
    wy4 = wvy[..., None] * jnp.maximum(1.0 - jnp.abs(ycb[..., None] - cy), 0.0)
    wy = wy4.sum(axis=2).reshape(PH * KB, N * H).astype(jnp.bfloat16)

    xc, wvx = _axis_coords(x1, PW, bin_w, W)
    kf = jax.lax.broadcasted_iota(jnp.int32, (PW, KB, 2), 1).astype(jnp.float32)
    xcb = xc + kf * float(W)
    cxk = cxk_ref[...].reshape(1, 1, 1, W * KB)
    bd4 = wvx[..., None] * jnp.maximum(1.0 - jnp.abs(xcb[..., None] - cxk), 0.0)
    bd = bd4.sum(axis=2).reshape(PW * KB, W * KB).astype(jnp.bfloat16)

    for x0 in range(W):
        chunk = jnp.dot(wy, feat_ref[:, pl.ds(x0 * C, C)],
                        preferred_element_type=jnp.float32)
        t2_ref[x0] = chunk.astype(jnp.bfloat16)

    for ph in range(PH):
        rhs = t2_ref[:, pl.ds(ph * KB, KB), :].reshape(W * KB, C)
        out_ph = jnp.dot(bd, rhs, preferred_element_type=jnp.float32)
        out_ref[ph] = out_ph.astype(jnp.bfloat16).reshape(PW, KB, C)


def kernel(input_tensor, rois):
    k = rois.shape[0]
    kp = ((k + KB - 1) // KB) * KB
    if kp != k:
        rois = jnp.concatenate(
            [rois, jnp.zeros((kp - k, 5), rois.dtype)], axis=0)
    featT = jnp.transpose(input_tensor, (0, 2, 3, 1)).reshape(
        N * H, W * C).astype(jnp.bfloat16)
    grid = (kp // KB,)
    cy = jnp.arange(N * H, dtype=jnp.float32).reshape(1, N * H)
    cxk = (jnp.arange(W * KB, dtype=jnp.float32) // KB
           + (jnp.arange(W * KB) % KB).astype(jnp.float32) * W
           ).reshape(1, W * KB)
    out = pl.pallas_call(
        _body,
        grid=grid,
        in_specs=[
            pl.BlockSpec((N * H, W * C), lambda i: (0, 0)),
            pl.BlockSpec((KB, 5), lambda i: (i, 0)),
            pl.BlockSpec((1, N * H), lambda i: (0, 0)),
            pl.BlockSpec((1, W * KB), lambda i: (0, 0)),
        ],
        out_specs=pl.BlockSpec((PH, PW, KB, C), lambda i: (0, 0, i, 0)),
        out_shape=jax.ShapeDtypeStruct((PH, PW, kp, C), jnp.bfloat16),
        scratch_shapes=[pltpu.VMEM((W, PH * KB, C), jnp.bfloat16)],
        compiler_params=pltpu.CompilerParams(
            dimension_semantics=("arbitrary",),
        ),
    )(featT, rois, cy, cxk)
    return jnp.transpose(out[:, :, :k], (2, 3, 0, 1)).astype(jnp.float32)
